# baseline (speedup 1.0000x reference)
import functools

import jax
import jax.numpy as jnp
from jax import lax
from jax.experimental import pallas as pl
from jax.experimental.pallas import tpu as pltpu
from jax.experimental.pallas import tpu_sc as plsc

_B = 2048
_IN = 8192
_OUT = 8192
_L = 16
_NC = 2
_NS = 16
_NW = _NC * _NS
_ROWS_PER_TILE = _B // _NW
_NQ = _ROWS_PER_TILE // 4
_H = _OUT // 2
_NGH = _H // _L
_JS = _OUT // _NS
_WC = _JS * 16 // 2


def _sc_body(x_hbm, w_hbm, ipk_hbm,
             out_hbm, cof_hbm,
             r0_v, r1_v, r2_v, r3_v, r4_v, r5_v, r6_v, r7_v,
             o00_v, o01_v, o10_v, o11_v, o20_v, o21_v, o30_v, o31_v,
             ipk_v, c01_v, c23_v,
             w_v, st0_v, sta_v,
             isem0, isem1, isem2, isem3, isem4, isem5, isem6, isem7,
             osem00, osem01, osem10, osem11,
             osem20, osem21, osem30, osem31,
             ipksem):
    c = lax.axis_index("c")
    s = lax.axis_index("s")
    wid = s * _NC + c
    base = wid * _ROWS_PER_TILE

    rows = (r0_v, r1_v, r2_v, r3_v, r4_v, r5_v, r6_v, r7_v)
    outs = ((o00_v, o01_v), (o10_v, o11_v), (o20_v, o21_v), (o30_v, o31_v))
    isems = (isem0, isem1, isem2, isem3, isem4, isem5, isem6, isem7)
    osems = ((osem00, osem01), (osem10, osem11),
             (osem20, osem21), (osem30, osem31))

    pltpu.async_copy(ipk_hbm, ipk_v, ipksem)
    for b in range(8):
        pltpu.async_copy(x_hbm.at[base + b], rows[b], isems[b])

    jbase = s * _JS
    half_groups = _JS // _L // 2

    def _cgroup(g, carry):
        j0 = g * _L
        loc = g * _L - (g // half_groups) * (half_groups * _L)
        stride = lax.iota(jnp.int32, _L) * 16
        cols = []
        for k in range(16):
            cols.append(plsc.load_gather(w_v, [loc * 16 + k + stride]))
        m = cols[0]
        for k in range(1, 16):
            m = jnp.maximum(m, cols[k])
        e = [jnp.exp(col - m) for col in cols]
        tot = e[0]
        for k in range(1, 16):
            tot = tot + e[k]
        inv = 1.0 / tot
        c0 = (e[8] + e[9] + e[10] + e[11]
              + e[12] + e[13] + e[14] + e[15]) * inv
        ca = (e[2] + e[3] + e[6] + e[7]
              - e[8] - e[9] - e[12] - e[13]) * inv
        cb = (e[4] + e[5] + e[6] + e[7]
              - e[8] - e[9] - e[10] - e[11]) * inv
        cab = (e[1] - e[2] - e[4] - 2.0 * e[6] - e[7]
               + e[8] + 2.0 * e[9] + e[11] + e[13] - e[14]) * inv
        pk01 = plsc.pack(c0, ca, format=plsc.PackFormat.INTERLEAVED)
        pk23 = plsc.pack(cb, cab, format=plsc.PackFormat.INTERLEAVED)
        st0_v[pl.ds(j0, _L)] = plsc.bitcast(pk01, jnp.float32)
        sta_v[pl.ds(j0, _L)] = plsc.bitcast(pk23, jnp.float32)
        return carry

    for chunk in range(2):
        pltpu.sync_copy(
            w_hbm.at[pl.ds(jbase * 16 + chunk * _WC, _WC)], w_v)
        lax.fori_loop(chunk * half_groups, (chunk + 1) * half_groups,
                      _cgroup, 0)

    pltpu.sync_copy(st0_v, cof_hbm.at[c, 0, pl.ds(jbase, _JS)])
    pltpu.sync_copy(sta_v, cof_hbm.at[c, 1, pl.ds(jbase, _JS)])
    plsc.subcore_barrier()
    pltpu.sync_copy(cof_hbm.at[c, 0], c01_v)
    pltpu.sync_copy(cof_hbm.at[c, 1], c23_v)

    pltpu.make_async_copy(ipk_hbm, ipk_v, ipksem).wait()

    def _quad(i, h):
        q = 2 * i + h
        bufs = tuple(rows[4 * h + k] for k in range(4))
        rx0, rx1, rx2, rx3 = bufs
        for k in range(4):
            pltpu.make_async_copy(x_hbm.at[base], bufs[k],
                                  isems[4 * h + k]).wait()

        for half in range(2):
            hbase = half * _H
            ob0, ob1, ob2, ob3 = (outs[r][half] for r in range(4))

            @pl.when(q >= 1)
            def _():
                for r in range(4):
                    pltpu.make_async_copy(
                        outs[r][half],
                        out_hbm.at[base, pl.ds(hbase, _H)],
                        osems[r][half]).wait()

            @plsc.parallel_loop(0, _NGH, unroll=6)
            def _g(g):
                loc = g * _L
                off = hbase + loc
                ipk = ipk_v[pl.ds(off, _L)]
                ia = lax.bitwise_and(ipk, jnp.int32(0xFFFF))
                ib = lax.shift_right_logical(ipk, jnp.int32(16))
                pk01 = plsc.bitcast(c01_v[pl.ds(off, _L)], jnp.bfloat16)
                pk23 = plsc.bitcast(c23_v[pl.ds(off, _L)], jnp.bfloat16)
                k0, ka = plsc.unpack(pk01, format=plsc.PackFormat.INTERLEAVED)
                kb, kab = plsc.unpack(pk23, format=plsc.PackFormat.INTERLEAVED)
                a0 = plsc.load_gather(rx0, [ia])
                b0 = plsc.load_gather(rx0, [ib])
                a1 = plsc.load_gather(rx1, [ia])
                b1 = plsc.load_gather(rx1, [ib])
                a2 = plsc.load_gather(rx2, [ia])
                b2 = plsc.load_gather(rx2, [ib])
                a3 = plsc.load_gather(rx3, [ia])
                b3 = plsc.load_gather(rx3, [ib])
                ob0[pl.ds(loc, _L)] = (k0 + ka * a0) + b0 * (kb + kab * a0)
                ob1[pl.ds(loc, _L)] = (k0 + ka * a1) + b1 * (kb + kab * a1)
                ob2[pl.ds(loc, _L)] = (k0 + ka * a2) + b2 * (kb + kab * a2)
                ob3[pl.ds(loc, _L)] = (k0 + ka * a3) + b3 * (kb + kab * a3)

            for r in range(4):
                pltpu.async_copy(
                    outs[r][half],
                    out_hbm.at[base + 4 * q + r, pl.ds(hbase, _H)],
                    osems[r][half])

        @pl.when(q < _NQ - 2)
        def _():
            for k in range(4):
                pltpu.async_copy(x_hbm.at[base + 4 * q + 8 + k],
                                 bufs[k], isems[4 * h + k])

    def _iter(i, carry):
        _quad(i, 0)
        _quad(i, 1)
        return carry

    lax.fori_loop(0, _NQ // 2, _iter, 0)

    for r in range(4):
        for half in range(2):
            pltpu.make_async_copy(outs[r][half],
                                  out_hbm.at[base, pl.ds(half * _H, _H)],
                                  osems[r][half]).wait()


_sc_main = functools.partial(
    pl.kernel,
    out_type=(jax.ShapeDtypeStruct((_B, _OUT), jnp.float32),
              jax.ShapeDtypeStruct((_NC, 2, _OUT), jnp.float32)),
    mesh=plsc.VectorSubcoreMesh(core_axis_name="c", subcore_axis_name="s"),
    compiler_params=pltpu.CompilerParams(needs_layout_passes=False),
    scratch_types=(
        [pltpu.VMEM((_IN,), jnp.float32)] * 8
        + [pltpu.VMEM((_H,), jnp.float32)] * 8
        + [
            pltpu.VMEM((_OUT,), jnp.int32),
            pltpu.VMEM((_OUT,), jnp.float32),
            pltpu.VMEM((_OUT,), jnp.float32),
            pltpu.VMEM((_WC,), jnp.float32),
            pltpu.VMEM((_JS,), jnp.float32),
            pltpu.VMEM((_JS,), jnp.float32),
        ]
        + [pltpu.SemaphoreType.DMA] * 8
        + [pltpu.SemaphoreType.DMA] * 8
        + [pltpu.SemaphoreType.DMA]
    ),
)(_sc_body)


def kernel(x, weights, idx_a, idx_b):
    ipk = jnp.bitwise_or(idx_a.astype(jnp.int32),
                         jnp.left_shift(idx_b.astype(jnp.int32), 16))
    out, _ = _sc_main(x, weights.reshape(-1), ipk)
    return out


# speedup vs baseline: 1.0045x; 1.0045x over previous
import functools

import jax
import jax.numpy as jnp
from jax import lax
from jax.experimental import pallas as pl
from jax.experimental.pallas import tpu as pltpu
from jax.experimental.pallas import tpu_sc as plsc

_B = 2048
_IN = 8192
_OUT = 8192
_L = 16
_NC = 2
_NS = 16
_NW = _NC * _NS
_ROWS_PER_TILE = _B // _NW
_NQ = _ROWS_PER_TILE // 4
_H = _OUT // 2
_NGH = _H // _L
_JS = _OUT // _NS
_WC = _JS * 16 // 2


def _sc_body(x_hbm, w_hbm, ipk_hbm,
             out_hbm, cof_hbm,
             r0_v, r1_v, r2_v, r3_v, r4_v, r5_v, r6_v, r7_v,
             o00_v, o01_v, o10_v, o11_v, o20_v, o21_v, o30_v, o31_v,
             ipk_v, c01_v, c23_v,
             w_v, st0_v, sta_v,
             isem0, isem1, isem2, isem3, isem4, isem5, isem6, isem7,
             osem00, osem01, osem10, osem11,
             osem20, osem21, osem30, osem31,
             ipksem):
    c = lax.axis_index("c")
    s = lax.axis_index("s")
    wid = s * _NC + c
    base = wid * _ROWS_PER_TILE

    rows = (r0_v, r1_v, r2_v, r3_v, r4_v, r5_v, r6_v, r7_v)
    outs = ((o00_v, o01_v), (o10_v, o11_v), (o20_v, o21_v), (o30_v, o31_v))
    isems = (isem0, isem1, isem2, isem3, isem4, isem5, isem6, isem7)
    osems = ((osem00, osem01), (osem10, osem11),
             (osem20, osem21), (osem30, osem31))

    pltpu.async_copy(ipk_hbm, ipk_v, ipksem)
    for b in range(8):
        pltpu.async_copy(x_hbm.at[base + b], rows[b], isems[b])

    jbase = s * _JS
    half_groups = _JS // _L // 2

    def _cgroup(g, carry):
        j0 = g * _L
        loc = g * _L - (g // half_groups) * (half_groups * _L)
        stride = lax.iota(jnp.int32, _L) * 16
        cols = []
        for k in range(16):
            cols.append(plsc.load_gather(w_v, [loc * 16 + k + stride]))
        m = cols[0]
        for k in range(1, 16):
            m = jnp.maximum(m, cols[k])
        e = [jnp.exp(col - m) for col in cols]
        tot = e[0]
        for k in range(1, 16):
            tot = tot + e[k]
        inv = 1.0 / tot
        c0 = (e[8] + e[9] + e[10] + e[11]
              + e[12] + e[13] + e[14] + e[15]) * inv
        ca = (e[2] + e[3] + e[6] + e[7]
              - e[8] - e[9] - e[12] - e[13]) * inv
        cb = (e[4] + e[5] + e[6] + e[7]
              - e[8] - e[9] - e[10] - e[11]) * inv
        cab = (e[1] - e[2] - e[4] - 2.0 * e[6] - e[7]
               + e[8] + 2.0 * e[9] + e[11] + e[13] - e[14]) * inv
        pk01 = plsc.pack(c0, ca, format=plsc.PackFormat.INTERLEAVED)
        pk23 = plsc.pack(cb, cab, format=plsc.PackFormat.INTERLEAVED)
        st0_v[pl.ds(j0, _L)] = plsc.bitcast(pk01, jnp.float32)
        sta_v[pl.ds(j0, _L)] = plsc.bitcast(pk23, jnp.float32)
        return carry

    for chunk in range(2):
        pltpu.sync_copy(
            w_hbm.at[pl.ds(jbase * 16 + chunk * _WC, _WC)], w_v)
        lax.fori_loop(chunk * half_groups, (chunk + 1) * half_groups,
                      _cgroup, 0)

    pltpu.sync_copy(st0_v, cof_hbm.at[c, 0, pl.ds(jbase, _JS)])
    pltpu.sync_copy(sta_v, cof_hbm.at[c, 1, pl.ds(jbase, _JS)])
    plsc.subcore_barrier()
    pltpu.sync_copy(cof_hbm.at[c, 0], c01_v)
    pltpu.sync_copy(cof_hbm.at[c, 1], c23_v)

    pltpu.make_async_copy(ipk_hbm, ipk_v, ipksem).wait()

    def _quad(i, h):
        q = 2 * i + h
        bufs = tuple(rows[4 * h + k] for k in range(4))
        rx0, rx1, rx2, rx3 = bufs
        for k in range(4):
            pltpu.make_async_copy(x_hbm.at[base], bufs[k],
                                  isems[4 * h + k]).wait()

        for half in range(2):
            hbase = half * _H
            ob0, ob1, ob2, ob3 = (outs[r][half] for r in range(4))

            @pl.when(q >= 1)
            def _():
                for r in range(4):
                    pltpu.make_async_copy(
                        outs[r][half],
                        out_hbm.at[base, pl.ds(hbase, _H)],
                        osems[r][half]).wait()

            @plsc.parallel_loop(0, _NGH, unroll=4)
            def _g(g):
                loc = g * _L
                off = hbase + loc
                ipk = ipk_v[pl.ds(off, _L)]
                ia = lax.bitwise_and(ipk, jnp.int32(0xFFFF))
                ib = lax.shift_right_logical(ipk, jnp.int32(16))
                pk01 = plsc.bitcast(c01_v[pl.ds(off, _L)], jnp.bfloat16)
                pk23 = plsc.bitcast(c23_v[pl.ds(off, _L)], jnp.bfloat16)
                k0, ka = plsc.unpack(pk01, format=plsc.PackFormat.INTERLEAVED)
                kb, kab = plsc.unpack(pk23, format=plsc.PackFormat.INTERLEAVED)
                a0 = plsc.load_gather(rx0, [ia])
                b0 = plsc.load_gather(rx0, [ib])
                a1 = plsc.load_gather(rx1, [ia])
                b1 = plsc.load_gather(rx1, [ib])
                a2 = plsc.load_gather(rx2, [ia])
                b2 = plsc.load_gather(rx2, [ib])
                a3 = plsc.load_gather(rx3, [ia])
                b3 = plsc.load_gather(rx3, [ib])
                ob0[pl.ds(loc, _L)] = (k0 + ka * a0) + b0 * (kb + kab * a0)
                ob1[pl.ds(loc, _L)] = (k0 + ka * a1) + b1 * (kb + kab * a1)
                ob2[pl.ds(loc, _L)] = (k0 + ka * a2) + b2 * (kb + kab * a2)
                ob3[pl.ds(loc, _L)] = (k0 + ka * a3) + b3 * (kb + kab * a3)

            for r in range(4):
                pltpu.async_copy(
                    outs[r][half],
                    out_hbm.at[base + 4 * q + r, pl.ds(hbase, _H)],
                    osems[r][half])

        @pl.when(q < _NQ - 2)
        def _():
            for k in range(4):
                pltpu.async_copy(x_hbm.at[base + 4 * q + 8 + k],
                                 bufs[k], isems[4 * h + k])

    def _iter(i, carry):
        _quad(i, 0)
        _quad(i, 1)
        return carry

    lax.fori_loop(0, _NQ // 2, _iter, 0)

    for r in range(4):
        for half in range(2):
            pltpu.make_async_copy(outs[r][half],
                                  out_hbm.at[base, pl.ds(half * _H, _H)],
                                  osems[r][half]).wait()


_sc_main = functools.partial(
    pl.kernel,
    out_type=(jax.ShapeDtypeStruct((_B, _OUT), jnp.float32),
              jax.ShapeDtypeStruct((_NC, 2, _OUT), jnp.float32)),
    mesh=plsc.VectorSubcoreMesh(core_axis_name="c", subcore_axis_name="s"),
    compiler_params=pltpu.CompilerParams(needs_layout_passes=False),
    scratch_types=(
        [pltpu.VMEM((_IN,), jnp.float32)] * 8
        + [pltpu.VMEM((_H,), jnp.float32)] * 8
        + [
            pltpu.VMEM((_OUT,), jnp.int32),
            pltpu.VMEM((_OUT,), jnp.float32),
            pltpu.VMEM((_OUT,), jnp.float32),
            pltpu.VMEM((_WC,), jnp.float32),
            pltpu.VMEM((_JS,), jnp.float32),
            pltpu.VMEM((_JS,), jnp.float32),
        ]
        + [pltpu.SemaphoreType.DMA] * 8
        + [pltpu.SemaphoreType.DMA] * 8
        + [pltpu.SemaphoreType.DMA]
    ),
)(_sc_body)


def kernel(x, weights, idx_a, idx_b):
    ipk = jnp.bitwise_or(idx_a.astype(jnp.int32),
                         jnp.left_shift(idx_b.astype(jnp.int32), 16))
    out, _ = _sc_main(x, weights.reshape(-1), ipk)
    return out
